# baseline (speedup 1.0000x reference)
import functools

import jax
import jax.numpy as jnp
from jax import lax
from jax.experimental import pallas as pl
from jax.experimental.pallas import tpu as pltpu
from jax.experimental.pallas import tpu_sc as plsc

_NC = 2
_NS = 16
_NW = _NC * _NS
_L = 16
_GP = 16


@functools.lru_cache(maxsize=None)
def _build(B, C, H, W, N, P):
    NPTS = N * P
    assert N % (2 * _GP) == 0
    NPAIR = N // (2 * _GP)
    max_rois = -(-NPAIR // _NW) * 2 * _GP
    NSL = min(N, max_rois + 16)
    assert NSL % 8 == 0 and (N - NSL) % 8 == 0

    mesh = plsc.VectorSubcoreMesh(
        core_axis_name="c", subcore_axis_name="s",
        num_cores=_NC, num_subcores=_NS)

    def body(feat_hbm, rois_hbm, off_hbm, invs_hbm, out_hbm,
             rois_v, off_v, invs_v,
             idx_a, idx_b, rows_a, rows_b, out_a, out_b,
             gsem_a, gsem_b, osem_a, osem_b):
        wid = lax.axis_index("s") * _NC + lax.axis_index("c")
        pstart = wid * NPAIR // _NW
        pend = (wid + 1) * NPAIR // _NW
        nbase = jnp.minimum(pstart * 2 * _GP, N - NSL)

        pltpu.sync_copy(rois_hbm.at[pl.ds(5 * nbase, 5 * NSL)], rois_v)
        pltpu.sync_copy(off_hbm.at[pl.ds(2 * P * nbase, 2 * P * NSL)],
                        off_v)
        pltpu.sync_copy(invs_hbm, invs_v)
        inv = invs_v[...]

        def fl(x):
            xf = x.astype(jnp.int32).astype(jnp.float32)
            return jnp.where(xf > x, xf - 1.0, xf)

        def make_indices(g, p, idx_ref):
            n = g * _GP + lax.iota(jnp.int32, _L)
            nl = jnp.minimum(n - nbase, NSL - 1)
            q5 = 5 * nl
            bf = plsc.load_gather(rois_v, [q5])
            x1 = plsc.load_gather(rois_v, [q5 + 1])
            y1 = plsc.load_gather(rois_v, [q5 + 2])
            x2 = plsc.load_gather(rois_v, [q5 + 3])
            y2 = plsc.load_gather(rois_v, [q5 + 4])
            q18 = 2 * P * nl + 2 * p
            ox = plsc.load_gather(off_v, [q18])
            oy = plsc.load_gather(off_v, [q18 + 1])
            cx = (x1 + x2) * 0.5
            cy = (y1 + y2) * 0.5
            bw = x2 - x1 + 1.0
            bh = y2 - y1 + 1.0
            sx = (cx + ox * bw * 0.1) * inv
            sy = (cy + oy * bh * 0.1) * inv
            hlf = jnp.clip(fl(sy), 0.0, H - 1.0)
            chi = hlf >= H - 1.0
            hhf = jnp.where(chi, hlf, hlf + 1.0)
            sy = jnp.where(chi, hlf, sy)
            wlf = jnp.clip(fl(sx), 0.0, W - 1.0)
            cwi = wlf >= W - 1.0
            whf = jnp.where(cwi, wlf, wlf + 1.0)
            sx = jnp.where(cwi, wlf, sx)
            lh = sy - hlf
            lw = sx - wlf
            uh = 1.0 - lh
            uw = 1.0 - lw
            bi = bf.astype(jnp.int32)
            rlo = (bi * H + hlf.astype(jnp.int32)) * W
            rhi = (bi * H + hhf.astype(jnp.int32)) * W
            wli = wlf.astype(jnp.int32)
            idx_ref[pl.ds(0, _L)] = rlo + wli
            idx_ref[pl.ds(_L, _L)] = rhi + wli
            return (uh * uw, uh * lw, lh * uw, lh * lw)

        dnums = lax.GatherDimensionNumbers(
            offset_dims=(), collapsed_slice_dims=(0,),
            start_index_map=(0,))

        def lanebcast(v, j):
            jv = jnp.full((_L, 1), j, jnp.int32)
            return lax.gather(v, jv, dnums, (1,),
                              mode=lax.GatherScatterMode.PROMISE_IN_BOUNDS)

        def combine(ws, rows_ref, out_ref):
            w1v, w2v, w3v, w4v = ws
            for j in range(_GP):
                w16 = [lanebcast(w, j) for w in (w1v, w2v, w3v, w4v)]
                w1, w2, w3, w4 = [
                    plsc.pack(w, w, format=plsc.PackFormat.INTERLEAVED)
                    for w in w16]
                for k in range(C // 32):
                    sl = pl.ds(k * _L, _L)
                    sr = pl.ds(C // 2 + k * _L, _L)
                    bc = lambda v: plsc.bitcast(v, jnp.bfloat16)
                    v1 = bc(rows_ref[j, sl])
                    v2 = bc(rows_ref[j, sr])
                    v3 = bc(rows_ref[_GP + j, sl])
                    v4 = bc(rows_ref[_GP + j, sr])
                    acc = v1 * w1 + v2 * w2 + v3 * w3 + v4 * w4
                    out_ref[j, sl] = plsc.bitcast(acc, jnp.int32)

        def fire(idx_ref, rows_ref, sem):
            return pltpu.async_copy(feat_hbm.at[idx_ref], rows_ref, sem)

        def block_body(p, carry):
            obase = p * N
            ws_first = make_indices(2 * pstart, p, idx_a)
            fire(idx_a, rows_a, gsem_a)

            def pbody(q, ws_a):
                ws_b = make_indices(2 * q + 1, p, idx_b)
                fire(idx_b, rows_b, gsem_b)
                pltpu.make_async_copy(
                    feat_hbm.at[idx_a], rows_a, gsem_a).wait()

                @pl.when(q > pstart)
                def _():
                    pltpu.make_async_copy(
                        out_a, out_hbm.at[pl.ds(0, _GP)], osem_a).wait()
                combine(ws_a, rows_a, out_a)
                pltpu.async_copy(
                    out_a, out_hbm.at[pl.ds(obase + 2 * q * _GP, _GP)],
                    osem_a)

                ws_a2 = make_indices(2 * (q + 1), p, idx_a)

                @pl.when(q + 1 < pend)
                def _():
                    fire(idx_a, rows_a, gsem_a)
                pltpu.make_async_copy(
                    feat_hbm.at[idx_b], rows_b, gsem_b).wait()

                @pl.when(q > pstart)
                def _():
                    pltpu.make_async_copy(
                        out_b, out_hbm.at[pl.ds(0, _GP)], osem_b).wait()
                combine(ws_b, rows_b, out_b)
                pltpu.async_copy(
                    out_b,
                    out_hbm.at[pl.ds(obase + (2 * q + 1) * _GP, _GP)],
                    osem_b)
                return ws_a2

            lax.fori_loop(pstart, pend, pbody, ws_first)

            pltpu.make_async_copy(out_a, out_hbm.at[pl.ds(0, _GP)],
                                  osem_a).wait()
            pltpu.make_async_copy(out_b, out_hbm.at[pl.ds(0, _GP)],
                                  osem_b).wait()
            return carry

        lax.fori_loop(0, P, block_body, 0)

    return pl.kernel(
        body,
        out_type=jax.ShapeDtypeStruct((NPTS, C // 2), jnp.int32),
        mesh=mesh,
        compiler_params=pltpu.CompilerParams(needs_layout_passes=False),
        scratch_types=[
            pltpu.VMEM((5 * NSL,), jnp.float32),
            pltpu.VMEM((2 * P * NSL,), jnp.float32),
            pltpu.VMEM((_L,), jnp.float32),
            pltpu.VMEM((2 * _L,), jnp.int32),
            pltpu.VMEM((2 * _L,), jnp.int32),
            pltpu.VMEM((2 * _GP, C), jnp.int32),
            pltpu.VMEM((2 * _GP, C), jnp.int32),
            pltpu.VMEM((_GP, C // 2), jnp.int32),
            pltpu.VMEM((_GP, C // 2), jnp.int32),
            pltpu.SemaphoreType.DMA,
            pltpu.SemaphoreType.DMA,
            pltpu.SemaphoreType.DMA,
            pltpu.SemaphoreType.DMA,
        ],
    )


_HB = 8


@functools.lru_cache(maxsize=None)
def _build_tpose(B, C, H, W):
    Ch = C // 2

    def tbody(x_ref, o_ref):
        x = x_ref[0]
        for h in range(_HB):
            sl = x[:, h, :]
            lo = jnp.transpose(sl[:Ch, :]).astype(jnp.bfloat16)
            hi = jnp.transpose(sl[Ch:, :]).astype(jnp.bfloat16)
            ulo = lax.bitcast_convert_type(lo, jnp.uint16).astype(jnp.uint32)
            uhi = lax.bitcast_convert_type(hi, jnp.uint16).astype(jnp.uint32)
            o_ref[0, h] = lax.bitcast_convert_type(
                ulo | (uhi << 16), jnp.int32)

    return pl.pallas_call(
        tbody,
        grid=(B, H // _HB),
        in_specs=[pl.BlockSpec((1, C, _HB, W), lambda b, h: (b, 0, h, 0))],
        out_specs=pl.BlockSpec((1, _HB, W, Ch), lambda b, h: (b, h, 0, 0)),
        out_shape=jax.ShapeDtypeStruct((B, H, W, Ch), jnp.int32),
    )


def kernel(feat_map, rois, offset, stride, num_point):
    B, C, H, W = feat_map.shape
    N = rois.shape[0]
    P = offset.shape[1] // 2
    assert C % 32 == 0
    t1 = _build_tpose(B, C, H, W)(feat_map).reshape(B * H * W, C // 2)
    tsh = jnp.concatenate([t1[1:], t1[:1]], axis=0)
    feat_t = jnp.concatenate([t1, tsh], axis=1)
    inv = jnp.full((_L,), 1.0, jnp.float32) / jnp.asarray(
        stride, jnp.float32)
    out_flat = _build(B, C, H, W, N, P)(
        feat_t, rois.reshape(-1), offset.reshape(-1), inv)
    ow = lax.bitcast_convert_type(out_flat, jnp.uint32)
    lo = lax.bitcast_convert_type(ow << 16, jnp.float32)
    hi = lax.bitcast_convert_type(ow & jnp.uint32(0xFFFF0000),
                                  jnp.float32)
    out_pnc = jnp.concatenate([lo, hi], axis=-1).reshape(P, N, C)
    return jnp.transpose(out_pnc, (1, 0, 2))


# speedup vs baseline: 1.4127x; 1.4127x over previous
import functools

import jax
import jax.numpy as jnp
from jax import lax
from jax.experimental import pallas as pl
from jax.experimental.pallas import tpu as pltpu
from jax.experimental.pallas import tpu_sc as plsc

_NC = 2
_NS = 16
_NW = _NC * _NS
_L = 16
_GP = 16


@functools.lru_cache(maxsize=None)
def _build(B, C, H, W, N, P):
    NPTS = N * P
    assert N % (2 * _GP) == 0
    NPAIR = N // (2 * _GP)
    max_rois = -(-NPAIR // _NW) * 2 * _GP
    NSL = min(N, max_rois + 16)
    assert NSL % 8 == 0 and (N - NSL) % 8 == 0

    mesh = plsc.VectorSubcoreMesh(
        core_axis_name="c", subcore_axis_name="s",
        num_cores=_NC, num_subcores=_NS)

    def body(feat_hbm, rois_hbm, off_hbm, invs_hbm, out_hbm,
             rois_v, off_v, invs_v,
             idx_a, idx_b, rows_a, rows_b, out_a, out_b,
             gsem_a, gsem_b, osem_a, osem_b):
        wid = lax.axis_index("s") * _NC + lax.axis_index("c")
        pstart = wid * NPAIR // _NW
        pend = (wid + 1) * NPAIR // _NW
        nbase = jnp.minimum(pstart * 2 * _GP, N - NSL)

        pltpu.sync_copy(rois_hbm.at[pl.ds(5 * nbase, 5 * NSL)], rois_v)
        pltpu.sync_copy(off_hbm.at[pl.ds(2 * P * nbase, 2 * P * NSL)],
                        off_v)
        pltpu.sync_copy(invs_hbm, invs_v)
        inv = invs_v[...]

        def fl(x):
            xf = x.astype(jnp.int32).astype(jnp.float32)
            return jnp.where(xf > x, xf - 1.0, xf)

        def make_indices(g, p, idx_ref):
            n = g * _GP + lax.iota(jnp.int32, _L)
            nl = jnp.minimum(n - nbase, NSL - 1)
            q5 = 5 * nl
            bf = plsc.load_gather(rois_v, [q5])
            x1 = plsc.load_gather(rois_v, [q5 + 1])
            y1 = plsc.load_gather(rois_v, [q5 + 2])
            x2 = plsc.load_gather(rois_v, [q5 + 3])
            y2 = plsc.load_gather(rois_v, [q5 + 4])
            q18 = 2 * P * nl + 2 * p
            ox = plsc.load_gather(off_v, [q18])
            oy = plsc.load_gather(off_v, [q18 + 1])
            cx = (x1 + x2) * 0.5
            cy = (y1 + y2) * 0.5
            bw = x2 - x1 + 1.0
            bh = y2 - y1 + 1.0
            sx = (cx + ox * bw * 0.1) * inv
            sy = (cy + oy * bh * 0.1) * inv
            hlf = jnp.clip(fl(sy), 0.0, H - 1.0)
            chi = hlf >= H - 1.0
            hhf = jnp.where(chi, hlf, hlf + 1.0)
            sy = jnp.where(chi, hlf, sy)
            wlf = jnp.clip(fl(sx), 0.0, W - 1.0)
            cwi = wlf >= W - 1.0
            whf = jnp.where(cwi, wlf, wlf + 1.0)
            sx = jnp.where(cwi, wlf, sx)
            lh = sy - hlf
            lw = sx - wlf
            uh = 1.0 - lh
            uw = 1.0 - lw
            bi = bf.astype(jnp.int32)
            rlo = (bi * H + hlf.astype(jnp.int32)) * W
            rhi = (bi * H + hhf.astype(jnp.int32)) * W
            wli = wlf.astype(jnp.int32)
            whi = whf.astype(jnp.int32)
            idx_ref[pl.ds(0, _L)] = rlo + wli
            idx_ref[pl.ds(_L, _L)] = rlo + whi
            idx_ref[pl.ds(2 * _L, _L)] = rhi + wli
            idx_ref[pl.ds(3 * _L, _L)] = rhi + whi
            return (uh * uw, uh * lw, lh * uw, lh * lw)

        dnums = lax.GatherDimensionNumbers(
            offset_dims=(), collapsed_slice_dims=(0,),
            start_index_map=(0,))

        def lanebcast(v, j):
            jv = jnp.full((_L, 1), j, jnp.int32)
            return lax.gather(v, jv, dnums, (1,),
                              mode=lax.GatherScatterMode.PROMISE_IN_BOUNDS)

        def combine(ws, rows_ref, out_ref):
            w1v, w2v, w3v, w4v = ws
            for j in range(_GP):
                w16 = [lanebcast(w, j) for w in (w1v, w2v, w3v, w4v)]
                w1, w2, w3, w4 = [
                    plsc.pack(w, w, format=plsc.PackFormat.INTERLEAVED)
                    for w in w16]
                for k in range(C // 32):
                    sl = pl.ds(k * _L, _L)
                    bc = lambda v: plsc.bitcast(v, jnp.bfloat16)
                    v1 = bc(rows_ref[j, sl])
                    v2 = bc(rows_ref[_GP + j, sl])
                    v3 = bc(rows_ref[2 * _GP + j, sl])
                    v4 = bc(rows_ref[3 * _GP + j, sl])
                    acc = v1 * w1 + v2 * w2 + v3 * w3 + v4 * w4
                    alo, ahi = plsc.unpack(
                        acc, format=plsc.PackFormat.INTERLEAVED)
                    out_ref[2 * j, sl] = alo
                    out_ref[2 * j + 1, sl] = ahi

        def fire(idx_ref, rows_ref, sem):
            return pltpu.async_copy(feat_hbm.at[idx_ref], rows_ref, sem)

        def block_body(p, carry):
            obase = 2 * p * N
            ws_first = make_indices(2 * pstart, p, idx_a)
            fire(idx_a, rows_a, gsem_a)

            def pbody(q, ws_a):
                ws_b = make_indices(2 * q + 1, p, idx_b)
                fire(idx_b, rows_b, gsem_b)
                pltpu.make_async_copy(
                    feat_hbm.at[idx_a], rows_a, gsem_a).wait()

                @pl.when(q > pstart)
                def _():
                    pltpu.make_async_copy(
                        out_a, out_hbm.at[pl.ds(0, 2 * _GP)],
                        osem_a).wait()
                combine(ws_a, rows_a, out_a)
                pltpu.async_copy(
                    out_a,
                    out_hbm.at[pl.ds(obase + 4 * q * _GP, 2 * _GP)],
                    osem_a)

                ws_a2 = make_indices(2 * (q + 1), p, idx_a)

                @pl.when(q + 1 < pend)
                def _():
                    fire(idx_a, rows_a, gsem_a)
                pltpu.make_async_copy(
                    feat_hbm.at[idx_b], rows_b, gsem_b).wait()

                @pl.when(q > pstart)
                def _():
                    pltpu.make_async_copy(
                        out_b, out_hbm.at[pl.ds(0, 2 * _GP)],
                        osem_b).wait()
                combine(ws_b, rows_b, out_b)
                pltpu.async_copy(
                    out_b,
                    out_hbm.at[pl.ds(obase + (4 * q + 2) * _GP, 2 * _GP)],
                    osem_b)
                return ws_a2

            lax.fori_loop(pstart, pend, pbody, ws_first)

            pltpu.make_async_copy(out_a, out_hbm.at[pl.ds(0, 2 * _GP)],
                                  osem_a).wait()
            pltpu.make_async_copy(out_b, out_hbm.at[pl.ds(0, 2 * _GP)],
                                  osem_b).wait()
            return carry

        lax.fori_loop(0, P, block_body, 0)

    return pl.kernel(
        body,
        out_type=jax.ShapeDtypeStruct((2 * NPTS, C // 2), jnp.float32),
        mesh=mesh,
        compiler_params=pltpu.CompilerParams(needs_layout_passes=False),
        scratch_types=[
            pltpu.VMEM((5 * NSL,), jnp.float32),
            pltpu.VMEM((2 * P * NSL,), jnp.float32),
            pltpu.VMEM((_L,), jnp.float32),
            pltpu.VMEM((4 * _L,), jnp.int32),
            pltpu.VMEM((4 * _L,), jnp.int32),
            pltpu.VMEM((4 * _GP, C // 2), jnp.int32),
            pltpu.VMEM((4 * _GP, C // 2), jnp.int32),
            pltpu.VMEM((2 * _GP, C // 2), jnp.float32),
            pltpu.VMEM((2 * _GP, C // 2), jnp.float32),
            pltpu.SemaphoreType.DMA,
            pltpu.SemaphoreType.DMA,
            pltpu.SemaphoreType.DMA,
            pltpu.SemaphoreType.DMA,
        ],
    )


_HB = 8


@functools.lru_cache(maxsize=None)
def _build_tpose(B, C, H, W):
    Ch = C // 2

    def tbody(x_ref, o_ref):
        x = x_ref[0]
        for h in range(_HB):
            sl = x[:, h, :]
            lo = jnp.transpose(sl[:Ch, :]).astype(jnp.bfloat16)
            hi = jnp.transpose(sl[Ch:, :]).astype(jnp.bfloat16)
            ulo = lax.bitcast_convert_type(lo, jnp.uint16).astype(jnp.uint32)
            uhi = lax.bitcast_convert_type(hi, jnp.uint16).astype(jnp.uint32)
            o_ref[0, h] = lax.bitcast_convert_type(
                ulo | (uhi << 16), jnp.int32)

    return pl.pallas_call(
        tbody,
        grid=(B, H // _HB),
        in_specs=[pl.BlockSpec((1, C, _HB, W), lambda b, h: (b, 0, h, 0))],
        out_specs=pl.BlockSpec((1, _HB, W, Ch), lambda b, h: (b, h, 0, 0)),
        out_shape=jax.ShapeDtypeStruct((B, H, W, Ch), jnp.int32),
    )


def kernel(feat_map, rois, offset, stride, num_point):
    B, C, H, W = feat_map.shape
    N = rois.shape[0]
    P = offset.shape[1] // 2
    assert C % 32 == 0
    feat_t = _build_tpose(B, C, H, W)(feat_map).reshape(B * H * W, C // 2)
    inv = jnp.full((_L,), 1.0, jnp.float32) / jnp.asarray(
        stride, jnp.float32)
    out_flat = _build(B, C, H, W, N, P)(
        feat_t, rois.reshape(-1), offset.reshape(-1), inv)
    out_pnc = out_flat.reshape(P, N, C)
    return jnp.transpose(out_pnc, (1, 0, 2))
